# baseline (speedup 1.0000x reference)
import functools

import jax
import jax.numpy as jnp
from jax import lax
from jax.experimental import pallas as pl
from jax.experimental.pallas import tpu as pltpu
from jax.experimental.pallas import tpu_sc as plsc

N_NODES = 10000
N_PAD = 10240
HALF = N_PAD // 2
N_EDGES = 320000
NUM_GRAPHS = 64
NUM_CORES = 2
NUM_SUBCORES = 16
NW = NUM_CORES * NUM_SUBCORES
BLK = 128
NBLK = 80
E_PAD = NW * NBLK * BLK
NBLK_P = E_PAD // BLK
NBLK_T = NBLK_P // NUM_SUBCORES
ROWS = N_PAD // NUM_SUBCORES
ROWS_P = HALF // NUM_SUBCORES
BM = 1024


def _mesh():
    return plsc.VectorSubcoreMesh(core_axis_name="c", subcore_axis_name="s")


_SC_PARAMS = pltpu.CompilerParams(use_tc_tiling_on_sc=False,
                                  needs_layout_passes=False)


EPT = E_PAD // NW
NVEC = EPT // 16


def _make_deg():
    def body(dst_hbm, out_hbm, dst_v, hist):
        c = lax.axis_index("c")
        s = lax.axis_index("s")
        wid = c * NUM_SUBCORES + s
        pltpu.sync_copy(dst_hbm.at[wid], dst_v)
        zeros = jnp.zeros((16,), jnp.float32)

        def zstep(i, carry):
            hist[pl.ds(i * 16, 16)] = zeros
            return carry
        lax.fori_loop(0, N_PAD // 16, zstep, 0)
        ones = jnp.ones((16,), jnp.float32)

        def step(j, carry):
            plsc.addupdate_scatter(hist, [dst_v[j, :]], ones)
            return carry
        lax.fori_loop(0, NVEC, step, 0)
        pltpu.sync_copy(hist, out_hbm.at[wid])

    return pl.kernel(
        body,
        out_type=jax.ShapeDtypeStruct((NW, N_PAD), jnp.float32),
        mesh=_mesh(),
        compiler_params=_SC_PARAMS,
        scratch_types=[
            pltpu.VMEM((NVEC, 16), jnp.int32),
            pltpu.VMEM((N_PAD,), jnp.float32),
        ],
    )


def _make_agg_part():
    def body(src_hbm, dst_hbm, cnt_hbm, g_hbm, out_hbm,
             src_v, dst_v, cnt_v, buf_a, buf_b, acc, sem_a, sem_b):
        c = lax.axis_index("c")
        s = lax.axis_index("s")
        r0 = s * ROWS_P
        pltpu.sync_copy(g_hbm.at[pl.ds(c * HALF + r0, ROWS_P)],
                        acc.at[pl.ds(r0, ROWS_P)])
        pltpu.sync_copy(cnt_hbm.at[c], cnt_v)
        nblk = jnp.max(cnt_v[...])
        base = s * nblk
        pltpu.sync_copy(src_hbm.at[c, pl.ds(base, NBLK_T)], src_v)
        pltpu.sync_copy(dst_hbm.at[c, pl.ds(base, NBLK_T)], dst_v)
        plsc.subcore_barrier()

        nsteps = nblk // 2

        @pl.when(nsteps > 0)
        def _():
            pltpu.async_copy(g_hbm.at[src_v.at[0]], buf_a, sem_a)

        def step(i, carry):
            j = 2 * i
            pltpu.async_copy(g_hbm.at[src_v.at[j + 1]], buf_b, sem_b)
            pltpu.make_async_copy(g_hbm.at[src_v.at[j]], buf_a, sem_a).wait()
            pltpu.sync_copy(buf_a, acc.at[dst_v.at[j]], add=True)

            @pl.when(i < nsteps - 1)
            def _():
                pltpu.async_copy(g_hbm.at[src_v.at[j + 2]], buf_a, sem_a)

            pltpu.make_async_copy(g_hbm.at[src_v.at[j + 1]], buf_b, sem_b).wait()
            pltpu.sync_copy(buf_b, acc.at[dst_v.at[j + 1]], add=True)
            return carry
        lax.fori_loop(0, nsteps, step, 0)
        plsc.subcore_barrier()
        pltpu.sync_copy(acc.at[pl.ds(r0, ROWS_P)],
                        out_hbm.at[pl.ds(c * HALF + r0, ROWS_P)])

    return pl.kernel(
        body,
        out_type=jax.ShapeDtypeStruct((N_PAD, 128), jnp.float32),
        mesh=_mesh(),
        compiler_params=_SC_PARAMS,
        scratch_types=[
            pltpu.VMEM((NBLK_T, BLK), jnp.int32),
            pltpu.VMEM((NBLK_T, BLK), jnp.int32),
            pltpu.VMEM((16,), jnp.int32),
            pltpu.VMEM((BLK, 128), jnp.float32),
            pltpu.VMEM((BLK, 128), jnp.float32),
            pltpu.VMEM_SHARED((HALF, 128), jnp.float32),
            pltpu.SemaphoreType.DMA,
            pltpu.SemaphoreType.DMA,
        ],
    )


def _make_agg(feat):
    def body(src_hbm, dst_hbm, g_hbm, z_hbm, out_hbm,
             src_v, dst_v, buf_a, buf_b, acc, sem_a, sem_b):
        c = lax.axis_index("c")
        s = lax.axis_index("s")
        wid = c * NUM_SUBCORES + s
        r0 = s * ROWS

        @pl.when(c == 0)
        def _():
            pltpu.sync_copy(g_hbm.at[pl.ds(r0, ROWS)], acc.at[pl.ds(r0, ROWS)])

        @pl.when(c != 0)
        def _():
            pltpu.sync_copy(z_hbm.at[pl.ds(r0, ROWS)], acc.at[pl.ds(r0, ROWS)])

        pltpu.sync_copy(src_hbm.at[wid], src_v)
        pltpu.sync_copy(dst_hbm.at[wid], dst_v)
        plsc.subcore_barrier()

        pltpu.async_copy(g_hbm.at[src_v.at[0]], buf_a, sem_a)

        def step(i, carry):
            j = 2 * i
            pltpu.async_copy(g_hbm.at[src_v.at[j + 1]], buf_b, sem_b)
            pltpu.make_async_copy(g_hbm.at[src_v.at[j]], buf_a, sem_a).wait()
            pltpu.sync_copy(buf_a, acc.at[dst_v.at[j]], add=True)

            @pl.when(i < NBLK // 2 - 1)
            def _():
                pltpu.async_copy(g_hbm.at[src_v.at[j + 2]], buf_a, sem_a)

            pltpu.make_async_copy(g_hbm.at[src_v.at[j + 1]], buf_b, sem_b).wait()
            pltpu.sync_copy(buf_b, acc.at[dst_v.at[j + 1]], add=True)
            return carry
        lax.fori_loop(0, NBLK // 2, step, 0)
        plsc.subcore_barrier()
        pltpu.sync_copy(acc.at[pl.ds(r0, ROWS)], out_hbm.at[c, pl.ds(r0, ROWS)])

    return pl.kernel(
        body,
        out_type=jax.ShapeDtypeStruct((NUM_CORES, N_PAD, feat), jnp.float32),
        mesh=_mesh(),
        compiler_params=_SC_PARAMS,
        scratch_types=[
            pltpu.VMEM((NBLK, BLK), jnp.int32),
            pltpu.VMEM((NBLK, BLK), jnp.int32),
            pltpu.VMEM((BLK, feat), jnp.float32),
            pltpu.VMEM((BLK, feat), jnp.float32),
            pltpu.VMEM_SHARED((N_PAD, feat), jnp.float32),
            pltpu.SemaphoreType.DMA,
            pltpu.SemaphoreType.DMA,
        ],
    )


def _tc_pre_body(degp_ref, x_ref, w_ref, dinv_ref, g_ref):
    dp = degp_ref[...]
    deg = jnp.sum(dp, axis=0)[:, None] + 1.0
    dinv = lax.rsqrt(deg)
    dinv_ref[...] = dinv
    g_ref[...] = dinv * jnp.dot(x_ref[...], w_ref[...],
                                preferred_element_type=jnp.float32)


_tc_pre = pl.pallas_call(
    _tc_pre_body,
    grid=(N_PAD // BM,),
    in_specs=[
        pl.BlockSpec((NW, BM), lambda i: (0, i)),
        pl.BlockSpec((BM, 128), lambda i: (i, 0)),
        pl.BlockSpec((128, 128), lambda i: (0, 0)),
    ],
    out_specs=[
        pl.BlockSpec((BM, 1), lambda i: (i, 0)),
        pl.BlockSpec((BM, 128), lambda i: (i, 0)),
    ],
    out_shape=[
        jax.ShapeDtypeStruct((N_PAD, 1), jnp.float32),
        jax.ShapeDtypeStruct((N_PAD, 128), jnp.float32),
    ],
)


def _row_mask(i):
    rid = lax.broadcasted_iota(jnp.int32, (BM, 1), 0) + i * BM
    return (rid < N_NODES).astype(jnp.float32)


def _make_tc_mid(fout):
    def body(p_ref, dinv_ref, b_ref, w_ref, g_ref):
        i = pl.program_id(0)
        p = p_ref[...]
        dinv = dinv_ref[...]
        h = jnp.maximum(dinv * p + b_ref[...], 0.0)
        g = dinv * jnp.dot(h, w_ref[...], preferred_element_type=jnp.float32)
        g_ref[...] = g * _row_mask(i)

    def index(i):
        return (i, 0)

    return pl.pallas_call(
        body,
        grid=(N_PAD // BM,),
        in_specs=[
            pl.BlockSpec((BM, 128), index),
            pl.BlockSpec((BM, 1), index),
            pl.BlockSpec((1, 128), lambda i: (0, 0)),
            pl.BlockSpec((128, fout), lambda i: (0, 0)),
        ],
        out_specs=pl.BlockSpec((BM, fout), index),
        out_shape=jax.ShapeDtypeStruct((N_PAD, fout), jnp.float32),
    )


def _tc_mid1_body(p_ref, dinv_ref, b_ref, w_ref, g_ref):
    i = pl.program_id(0)
    p = p_ref[...]
    dinv = dinv_ref[...]
    h = jnp.maximum(dinv * (p[0] + p[1]) + b_ref[...], 0.0)
    g = dinv * jnp.dot(h, w_ref[...], preferred_element_type=jnp.float32)
    g_ref[...] = g * _row_mask(i)


_tc_mid1 = pl.pallas_call(
    _tc_mid1_body,
    grid=(N_PAD // BM,),
    in_specs=[
        pl.BlockSpec((2, BM, 64), lambda i: (0, i, 0)),
        pl.BlockSpec((BM, 1), lambda i: (i, 0)),
        pl.BlockSpec((1, 64), lambda i: (0, 0)),
        pl.BlockSpec((64, 32), lambda i: (0, 0)),
    ],
    out_specs=pl.BlockSpec((BM, 32), lambda i: (i, 0)),
    out_shape=jax.ShapeDtypeStruct((N_PAD, 32), jnp.float32),
)


def _tc_final_body(p_ref, dinv_ref, b_ref, batch_ref, wfc_ref, bfc_ref, out_ref):
    p = p_ref[...]
    dinv = dinv_ref[...]
    h = jnp.maximum(dinv * (p[0] + p[1]) + b_ref[...], 0.0)
    ids = lax.broadcasted_iota(jnp.int32, (N_PAD, NUM_GRAPHS), 1)
    m = (batch_ref[...] == ids).astype(jnp.float32)
    pooled = lax.dot_general(m, h, (((0,), (0,)), ((), ())),
                             preferred_element_type=jnp.float32)
    counts = jnp.sum(m, axis=0)
    mean = pooled / jnp.maximum(counts, 1.0)[:, None]
    out_ref[...] = jnp.dot(mean, wfc_ref[...],
                           preferred_element_type=jnp.float32) + bfc_ref[...]


_tc_final = pl.pallas_call(
    _tc_final_body,
    out_shape=jax.ShapeDtypeStruct((NUM_GRAPHS, 10), jnp.float32),
)

_deg = _make_deg()
_agg_part = _make_agg_part()
_agg64 = _make_agg(64)
_agg32 = _make_agg(32)
_mid_128 = _make_tc_mid(128)
_mid_64 = _make_tc_mid(64)


def _partition_edges(src, dst):
    half = (dst < HALF).astype(jnp.int32)
    c1 = jnp.cumsum(half)
    c2 = jnp.cumsum(1 - half)
    n_lo = c1[-1]
    n_hi = N_EDGES - n_lo
    ii = jnp.arange(E_PAD, dtype=jnp.int32)
    e1 = jnp.clip(jnp.searchsorted(c1, ii + 1), 0, N_EDGES - 1)
    e2 = jnp.clip(jnp.searchsorted(c2, ii + 1), 0, N_EDGES - 1)
    v1 = ii < n_lo
    v2 = ii < n_hi
    src_lo = jnp.where(v1, src[e1], N_NODES)
    dst_lo = jnp.where(v1, dst[e1], 0)
    src_hi = jnp.where(v2, src[e2], N_NODES)
    dst_hi = jnp.where(v2, dst[e2] - HALF, 0)
    src2 = jnp.stack([src_lo, src_hi]).reshape(2, NBLK_P, BLK)
    dst2 = jnp.stack([dst_lo, dst_hi]).reshape(2, NBLK_P, BLK)

    def nblk(n):
        b = (n + NUM_SUBCORES * BLK - 1) // (NUM_SUBCORES * BLK)
        return ((b + 1) // 2) * 2
    cnt = jnp.stack([nblk(n_lo), nblk(n_hi)]).astype(jnp.int32)
    cnt_arr = jnp.broadcast_to(cnt[:, None], (2, 16))
    return src2, dst2, cnt_arr


def kernel(x, edge_index, batch, W1, b1, W2, b2, W3, b3, W4, b4, W5, b5,
           Wfc, bfc):
    src = edge_index[0].astype(jnp.int32)
    dst = edge_index[1].astype(jnp.int32)
    n_pad_e = E_PAD - N_EDGES
    pad_src = jnp.full((n_pad_e,), N_NODES, jnp.int32)
    pad_dst = N_NODES + (jnp.arange(n_pad_e, dtype=jnp.int32)
                         % (N_PAD - N_NODES))
    src_flat = jnp.concatenate([src, pad_src])
    dst_flat = jnp.concatenate([dst, pad_dst])
    src_r = src_flat.reshape(NW, NBLK, BLK)
    dst_r = dst_flat.reshape(NW, NBLK, BLK)
    dst_h = dst_flat.reshape(NW, NVEC, 16)
    src2, dst2, cnt_arr = _partition_edges(src, dst)
    x_p = jnp.zeros((N_PAD, 128), jnp.float32).at[:N_NODES].set(x)
    batch_p = jnp.full((N_PAD, 1), NUM_GRAPHS, jnp.int32)
    batch_p = batch_p.at[:N_NODES, 0].set(batch.astype(jnp.int32))
    z32 = jnp.zeros((N_PAD, 32), jnp.float32)
    z64 = jnp.zeros((N_PAD, 64), jnp.float32)

    degp = _deg(dst_h)
    dinv, g = _tc_pre(degp, x_p, W1)
    for b_prev, w_next in ((b1, W2), (b2, W3)):
        p = _agg_part(src2, dst2, cnt_arr, g)
        g = _mid_128(p, dinv, b_prev.reshape(1, -1), w_next)
    p = _agg_part(src2, dst2, cnt_arr, g)
    g64 = _mid_64(p, dinv, b3.reshape(1, -1), W4)
    p = _agg64(src_r, dst_r, g64, z64)
    g32 = _tc_mid1(p, dinv, b4.reshape(1, -1), W5)
    p32 = _agg32(src_r, dst_r, g32, z32)
    out = _tc_final(p32, dinv, b5.reshape(1, -1), batch_p, Wfc,
                    bfc.reshape(1, -1))
    return out


# speedup vs baseline: 25.3391x; 25.3391x over previous
import functools

import jax
import jax.numpy as jnp
from jax import lax
from jax.experimental import pallas as pl
from jax.experimental.pallas import tpu as pltpu
from jax.experimental.pallas import tpu_sc as plsc

N_NODES = 10000
N_PAD = 10240
N_EDGES = 320000
NUM_GRAPHS = 64
NUM_CORES = 2
NUM_SUBCORES = 16
NW = NUM_CORES * NUM_SUBCORES
BLK = 128
NBLK = 80
NBLK_D = 160
E_PAD = NW * NBLK * BLK
ROWS = N_PAD // NUM_SUBCORES
EPT = E_PAD // NW
NVEC = EPT // 16
BM = 1024


def _mesh():
    return plsc.VectorSubcoreMesh(core_axis_name="c", subcore_axis_name="s")


_SC_PARAMS = pltpu.CompilerParams(use_tc_tiling_on_sc=False,
                                  needs_layout_passes=False)


def _make_deg():
    def body(dst_hbm, out_hbm, dst_v, hist):
        c = lax.axis_index("c")
        s = lax.axis_index("s")
        wid = c * NUM_SUBCORES + s
        pltpu.sync_copy(dst_hbm.at[wid], dst_v)
        zeros = jnp.zeros((16,), jnp.float32)

        def zstep(i, carry):
            hist[pl.ds(i * 16, 16)] = zeros
            return carry
        lax.fori_loop(0, N_PAD // 16, zstep, 0)
        ones = jnp.ones((16,), jnp.float32)

        def step(j, carry):
            plsc.addupdate_scatter(hist, [dst_v[j, :]], ones)
            return carry
        lax.fori_loop(0, NVEC, step, 0)
        pltpu.sync_copy(hist, out_hbm.at[wid])

    return pl.kernel(
        body,
        out_type=jax.ShapeDtypeStruct((NW, N_PAD), jnp.float32),
        mesh=_mesh(),
        compiler_params=_SC_PARAMS,
        scratch_types=[
            pltpu.VMEM((NVEC, 16), jnp.int32),
            pltpu.VMEM((N_PAD,), jnp.float32),
        ],
    )


def _make_agg_dual():
    def body(src_hbm, dst_hbm, gs_hbm, out_hbm,
             src_v, dst_v, buf_a, buf_b, acc, sem_a, sem_b):
        c = lax.axis_index("c")
        s = lax.axis_index("s")
        r0 = s * ROWS
        pltpu.sync_copy(gs_hbm.at[c, pl.ds(r0, ROWS)], acc.at[pl.ds(r0, ROWS)])
        pltpu.sync_copy(src_hbm.at[s], src_v)
        pltpu.sync_copy(dst_hbm.at[s], dst_v)
        plsc.subcore_barrier()

        g_hbm = gs_hbm.at[c]
        pltpu.async_copy(g_hbm.at[src_v.at[0]], buf_a, sem_a)

        def step(i, carry):
            j = 2 * i
            pltpu.async_copy(g_hbm.at[src_v.at[j + 1]], buf_b, sem_b)
            pltpu.make_async_copy(g_hbm.at[src_v.at[j]], buf_a, sem_a).wait()
            pltpu.sync_copy(buf_a, acc.at[dst_v.at[j]], add=True)

            @pl.when(i < NBLK_D // 2 - 1)
            def _():
                pltpu.async_copy(g_hbm.at[src_v.at[j + 2]], buf_a, sem_a)

            pltpu.make_async_copy(g_hbm.at[src_v.at[j + 1]], buf_b, sem_b).wait()
            pltpu.sync_copy(buf_b, acc.at[dst_v.at[j + 1]], add=True)
            return carry
        lax.fori_loop(0, NBLK_D // 2, step, 0)
        plsc.subcore_barrier()
        pltpu.sync_copy(acc.at[pl.ds(r0, ROWS)], out_hbm.at[c, pl.ds(r0, ROWS)])

    return pl.kernel(
        body,
        out_type=jax.ShapeDtypeStruct((NUM_CORES, N_PAD, 64), jnp.float32),
        mesh=_mesh(),
        compiler_params=_SC_PARAMS,
        scratch_types=[
            pltpu.VMEM((NBLK_D, BLK), jnp.int32),
            pltpu.VMEM((NBLK_D, BLK), jnp.int32),
            pltpu.VMEM((BLK, 64), jnp.float32),
            pltpu.VMEM((BLK, 64), jnp.float32),
            pltpu.VMEM_SHARED((N_PAD, 64), jnp.float32),
            pltpu.SemaphoreType.DMA,
            pltpu.SemaphoreType.DMA,
        ],
    )


def _make_agg(feat):
    def body(src_hbm, dst_hbm, g_hbm, z_hbm, out_hbm,
             src_v, dst_v, buf_a, buf_b, acc, sem_a, sem_b):
        c = lax.axis_index("c")
        s = lax.axis_index("s")
        wid = c * NUM_SUBCORES + s
        r0 = s * ROWS

        @pl.when(c == 0)
        def _():
            pltpu.sync_copy(g_hbm.at[pl.ds(r0, ROWS)], acc.at[pl.ds(r0, ROWS)])

        @pl.when(c != 0)
        def _():
            pltpu.sync_copy(z_hbm.at[pl.ds(r0, ROWS)], acc.at[pl.ds(r0, ROWS)])

        pltpu.sync_copy(src_hbm.at[wid], src_v)
        pltpu.sync_copy(dst_hbm.at[wid], dst_v)
        plsc.subcore_barrier()

        pltpu.async_copy(g_hbm.at[src_v.at[0]], buf_a, sem_a)

        def step(i, carry):
            j = 2 * i
            pltpu.async_copy(g_hbm.at[src_v.at[j + 1]], buf_b, sem_b)
            pltpu.make_async_copy(g_hbm.at[src_v.at[j]], buf_a, sem_a).wait()
            pltpu.sync_copy(buf_a, acc.at[dst_v.at[j]], add=True)

            @pl.when(i < NBLK // 2 - 1)
            def _():
                pltpu.async_copy(g_hbm.at[src_v.at[j + 2]], buf_a, sem_a)

            pltpu.make_async_copy(g_hbm.at[src_v.at[j + 1]], buf_b, sem_b).wait()
            pltpu.sync_copy(buf_b, acc.at[dst_v.at[j + 1]], add=True)
            return carry
        lax.fori_loop(0, NBLK // 2, step, 0)
        plsc.subcore_barrier()
        pltpu.sync_copy(acc.at[pl.ds(r0, ROWS)], out_hbm.at[c, pl.ds(r0, ROWS)])

    return pl.kernel(
        body,
        out_type=jax.ShapeDtypeStruct((NUM_CORES, N_PAD, feat), jnp.float32),
        mesh=_mesh(),
        compiler_params=_SC_PARAMS,
        scratch_types=[
            pltpu.VMEM((NBLK, BLK), jnp.int32),
            pltpu.VMEM((NBLK, BLK), jnp.int32),
            pltpu.VMEM((BLK, feat), jnp.float32),
            pltpu.VMEM((BLK, feat), jnp.float32),
            pltpu.VMEM_SHARED((N_PAD, feat), jnp.float32),
            pltpu.SemaphoreType.DMA,
            pltpu.SemaphoreType.DMA,
        ],
    )


def _tc_pre_body(degp_ref, x_ref, w_ref, dinv_ref, gs_ref):
    dp = degp_ref[...]
    deg = jnp.sum(dp, axis=0)[:, None] + 1.0
    dinv = lax.rsqrt(deg)
    dinv_ref[...] = dinv
    g = dinv * jnp.dot(x_ref[...], w_ref[...],
                       preferred_element_type=jnp.float32)
    gs_ref[0, :, :] = g[:, :64]
    gs_ref[1, :, :] = g[:, 64:]


_tc_pre = pl.pallas_call(
    _tc_pre_body,
    grid=(N_PAD // BM,),
    in_specs=[
        pl.BlockSpec((NW, BM), lambda i: (0, i)),
        pl.BlockSpec((BM, 128), lambda i: (i, 0)),
        pl.BlockSpec((128, 128), lambda i: (0, 0)),
    ],
    out_specs=[
        pl.BlockSpec((BM, 1), lambda i: (i, 0)),
        pl.BlockSpec((2, BM, 64), lambda i: (0, i, 0)),
    ],
    out_shape=[
        jax.ShapeDtypeStruct((N_PAD, 1), jnp.float32),
        jax.ShapeDtypeStruct((2, N_PAD, 64), jnp.float32),
    ],
)


def _make_tc_mid2(fout):
    split = fout == 128

    def body(p_ref, dinv_ref, b_ref, w_ref, out_ref):
        p = p_ref[...]
        dinv = dinv_ref[...]
        agg = jnp.concatenate([p[0], p[1]], axis=-1)
        h = jnp.maximum(dinv * agg + b_ref[...], 0.0)
        g = dinv * jnp.dot(h, w_ref[...], preferred_element_type=jnp.float32)
        if split:
            out_ref[0, :, :] = g[:, :64]
            out_ref[1, :, :] = g[:, 64:]
        else:
            out_ref[...] = g

    if split:
        out_specs = pl.BlockSpec((2, BM, 64), lambda i: (0, i, 0))
        out_shape = jax.ShapeDtypeStruct((2, N_PAD, 64), jnp.float32)
    else:
        out_specs = pl.BlockSpec((BM, fout), lambda i: (i, 0))
        out_shape = jax.ShapeDtypeStruct((N_PAD, fout), jnp.float32)

    return pl.pallas_call(
        body,
        grid=(N_PAD // BM,),
        in_specs=[
            pl.BlockSpec((2, BM, 64), lambda i: (0, i, 0)),
            pl.BlockSpec((BM, 1), lambda i: (i, 0)),
            pl.BlockSpec((1, 128), lambda i: (0, 0)),
            pl.BlockSpec((128, fout), lambda i: (0, 0)),
        ],
        out_specs=out_specs,
        out_shape=out_shape,
    )


def _tc_mid1_body(p_ref, dinv_ref, b_ref, w_ref, g_ref):
    p = p_ref[...]
    dinv = dinv_ref[...]
    h = jnp.maximum(dinv * (p[0] + p[1]) + b_ref[...], 0.0)
    g_ref[...] = dinv * jnp.dot(h, w_ref[...],
                                preferred_element_type=jnp.float32)


_tc_mid1 = pl.pallas_call(
    _tc_mid1_body,
    grid=(N_PAD // BM,),
    in_specs=[
        pl.BlockSpec((2, BM, 64), lambda i: (0, i, 0)),
        pl.BlockSpec((BM, 1), lambda i: (i, 0)),
        pl.BlockSpec((1, 64), lambda i: (0, 0)),
        pl.BlockSpec((64, 32), lambda i: (0, 0)),
    ],
    out_specs=pl.BlockSpec((BM, 32), lambda i: (i, 0)),
    out_shape=jax.ShapeDtypeStruct((N_PAD, 32), jnp.float32),
)


def _tc_final_body(p_ref, dinv_ref, b_ref, batch_ref, wfc_ref, bfc_ref, out_ref):
    p = p_ref[...]
    dinv = dinv_ref[...]
    h = jnp.maximum(dinv * (p[0] + p[1]) + b_ref[...], 0.0)
    ids = lax.broadcasted_iota(jnp.int32, (N_PAD, NUM_GRAPHS), 1)
    m = (batch_ref[...] == ids).astype(jnp.float32)
    pooled = lax.dot_general(m, h, (((0,), (0,)), ((), ())),
                             preferred_element_type=jnp.float32)
    counts = jnp.sum(m, axis=0)
    mean = pooled / jnp.maximum(counts, 1.0)[:, None]
    out_ref[...] = jnp.dot(mean, wfc_ref[...],
                           preferred_element_type=jnp.float32) + bfc_ref[...]


_tc_final = pl.pallas_call(
    _tc_final_body,
    out_shape=jax.ShapeDtypeStruct((NUM_GRAPHS, 10), jnp.float32),
)

_deg = _make_deg()
_agg_dual = _make_agg_dual()
_agg64 = _make_agg(64)
_agg32 = _make_agg(32)
_mid2_128 = _make_tc_mid2(128)
_mid2_64 = _make_tc_mid2(64)


def kernel(x, edge_index, batch, W1, b1, W2, b2, W3, b3, W4, b4, W5, b5,
           Wfc, bfc):
    src = edge_index[0].astype(jnp.int32)
    dst = edge_index[1].astype(jnp.int32)
    n_pad_e = E_PAD - N_EDGES
    pad_src = jnp.zeros((n_pad_e,), jnp.int32)
    pad_dst = N_NODES + (jnp.arange(n_pad_e, dtype=jnp.int32)
                         % (N_PAD - N_NODES))
    src_flat = jnp.concatenate([src, pad_src])
    dst_flat = jnp.concatenate([dst, pad_dst])
    src_r = src_flat.reshape(NW, NBLK, BLK)
    dst_r = dst_flat.reshape(NW, NBLK, BLK)
    src_d = src_flat.reshape(NUM_SUBCORES, NBLK_D, BLK)
    dst_d = dst_flat.reshape(NUM_SUBCORES, NBLK_D, BLK)
    dst_h = dst_flat.reshape(NW, NVEC, 16)
    x_p = jnp.zeros((N_PAD, 128), jnp.float32).at[:N_NODES].set(x)
    batch_p = jnp.full((N_PAD, 1), NUM_GRAPHS, jnp.int32)
    batch_p = batch_p.at[:N_NODES, 0].set(batch.astype(jnp.int32))
    z32 = jnp.zeros((N_PAD, 32), jnp.float32)
    z64 = jnp.zeros((N_PAD, 64), jnp.float32)

    degp = _deg(dst_h)
    dinv, gs = _tc_pre(degp, x_p, W1)
    for b_prev, w_next in ((b1, W2), (b2, W3)):
        p = _agg_dual(src_d, dst_d, gs)
        gs = _mid2_128(p, dinv, b_prev.reshape(1, -1), w_next)
    p = _agg_dual(src_d, dst_d, gs)
    g64 = _mid2_64(p, dinv, b3.reshape(1, -1), W4)
    p = _agg64(src_r, dst_r, g64, z64)
    g32 = _tc_mid1(p, dinv, b4.reshape(1, -1), W5)
    p32 = _agg32(src_r, dst_r, g32, z32)
    out = _tc_final(p32, dinv, b5.reshape(1, -1), batch_p, Wfc,
                    bfc.reshape(1, -1))
    return out
